# baseline (speedup 1.0000x reference)
import jax
import jax.numpy as jnp
from jax.experimental import pallas as pl
from jax.experimental.pallas import tpu as pltpu

_N = 8192
_TA = 256
_NI = _N // _TA


def _loss_kernel(a_ref, bT_ref, p3T_ref, out_ref, colmin_ref):
    c = pl.program_id(0)
    i = pl.program_id(1)

    a = a_ref[0]
    bT = bT_ref[...]

    a2 = jnp.sum(a * a, axis=1, keepdims=True)
    b2 = jnp.sum(bT * bT, axis=0, keepdims=True)
    abf = a.astype(jnp.bfloat16).astype(jnp.float32)
    bbf = bT.astype(jnp.bfloat16).astype(jnp.float32)
    cross = abf[:, 0:1] * bbf[0:1, :]
    cross = cross + abf[:, 1:2] * bbf[1:2, :]
    cross = cross + abf[:, 2:3] * bbf[2:3, :]
    tile = jnp.maximum(a2 + b2 - 2.0 * cross, 0.0)

    w = jnp.where(c == 0, 0.5, 1.0)

    @pl.when(jnp.logical_and(c == 0, i == 0))
    def _init_out():
        diff = p3T_ref[...] - bT
        out_ref[0, 0] = jnp.sum(diff * diff) / (_N * 3)

    @pl.when(i == 0)
    def _init_colmin():
        colmin_ref[...] = jnp.full((1, _N), jnp.inf, jnp.float32)

    colmin_ref[...] = jnp.minimum(
        colmin_ref[...], jnp.min(tile, axis=0, keepdims=True)
    )

    row_min = jnp.min(tile, axis=1, keepdims=True)
    total = w * jnp.sum(jnp.sqrt(row_min)) / _N

    col_sum = jnp.sum(jnp.sqrt(colmin_ref[...])) / _N
    total = total + jnp.where(i == _NI - 1, w * col_sum, 0.0)

    out_ref[0, 0] = out_ref[0, 0] + total


def kernel(pc1, pc2):
    a01 = pc1[:2]
    bT = pc2.T
    p3T = pc1[3].T

    out = pl.pallas_call(
        _loss_kernel,
        grid=(2, _NI),
        in_specs=[
            pl.BlockSpec((1, _TA, 3), lambda c, i: (c, i, 0)),
            pl.BlockSpec((3, _N), lambda c, i: (0, 0)),
            pl.BlockSpec((3, _N), lambda c, i: (0, 0)),
        ],
        out_specs=pl.BlockSpec(memory_space=pltpu.SMEM),
        out_shape=jax.ShapeDtypeStruct((1, 1), jnp.float32),
        scratch_shapes=[pltpu.VMEM((1, _N), jnp.float32)],
        compiler_params=pltpu.CompilerParams(
            dimension_semantics=("arbitrary", "arbitrary"),
        ),
    )(a01, bT, p3T)
    return out[0, 0]


# speedup vs baseline: 2.8342x; 2.8342x over previous
import jax
import jax.numpy as jnp
from jax.experimental import pallas as pl
from jax.experimental.pallas import tpu as pltpu

_N = 8192
_TA = 256
_NI = _N // _TA


def _loss_kernel(a_ref, a8_ref, bT8_ref, bT_ref, p3T_ref, out_ref, colmin_ref):
    c = pl.program_id(0)
    i = pl.program_id(1)

    a = a_ref[0]
    a8 = a8_ref[0]
    bT8 = bT8_ref[...]
    bT = bT_ref[...]

    m = jax.lax.dot_general(
        a8, bT8,
        dimension_numbers=(((1,), (0,)), ((), ())),
        preferred_element_type=jnp.float32,
    )

    a2 = jnp.sum(a * a, axis=1, keepdims=True)
    b2 = jnp.sum(bT * bT, axis=0, keepdims=True)

    t = m + b2
    row_min = jnp.min(t, axis=1, keepdims=True) + a2
    row_min = jnp.maximum(row_min, 0.0)

    u = t + a2

    w = jnp.where(c == 0, 0.5, 1.0)

    @pl.when(jnp.logical_and(c == 0, i == 0))
    def _init_out():
        diff = p3T_ref[...] - bT
        out_ref[0, 0] = jnp.sum(diff * diff) / (_N * 3)

    @pl.when(i == 0)
    def _init_colmin():
        colmin_ref[...] = jnp.full((1, _N), jnp.inf, jnp.float32)

    colmin_ref[...] = jnp.minimum(
        colmin_ref[...], jnp.min(u, axis=0, keepdims=True)
    )

    total = w * jnp.sum(jnp.sqrt(row_min)) / _N

    col_final = jnp.maximum(colmin_ref[...], 0.0)
    col_sum = jnp.sum(jnp.sqrt(col_final)) / _N
    total = total + jnp.where(i == _NI - 1, w * col_sum, 0.0)

    out_ref[0, 0] = out_ref[0, 0] + total


def kernel(pc1, pc2):
    a01 = pc1[:2]
    a8 = jnp.zeros((2, _N, 8), jnp.bfloat16)
    a8 = a8.at[:, :, :3].set((-2.0 * a01).astype(jnp.bfloat16))
    bT = pc2.T
    bT8 = jnp.zeros((8, _N), jnp.bfloat16)
    bT8 = bT8.at[:3, :].set(bT.astype(jnp.bfloat16))
    p3T = pc1[3].T

    out = pl.pallas_call(
        _loss_kernel,
        grid=(2, _NI),
        in_specs=[
            pl.BlockSpec((1, _TA, 3), lambda c, i: (c, i, 0)),
            pl.BlockSpec((1, _TA, 8), lambda c, i: (c, i, 0)),
            pl.BlockSpec((8, _N), lambda c, i: (0, 0)),
            pl.BlockSpec((3, _N), lambda c, i: (0, 0)),
            pl.BlockSpec((3, _N), lambda c, i: (0, 0)),
        ],
        out_specs=pl.BlockSpec(memory_space=pltpu.SMEM),
        out_shape=jax.ShapeDtypeStruct((1, 1), jnp.float32),
        scratch_shapes=[pltpu.VMEM((1, _N), jnp.float32)],
        compiler_params=pltpu.CompilerParams(
            dimension_semantics=("arbitrary", "arbitrary"),
        ),
    )(a01, a8, bT8, bT, p3T)
    return out[0, 0]


# speedup vs baseline: 3.2061x; 1.1312x over previous
import jax
import jax.numpy as jnp
from jax.experimental import pallas as pl
from jax.experimental.pallas import tpu as pltpu

_N = 8192
_TA = 512
_NI = _N // _TA


def _loss_kernel(a_ref, a8_ref, bT8_ref, bT_ref, p3T_ref, out_ref, colmin_ref):
    c = pl.program_id(0)
    i = pl.program_id(1)

    a = a_ref[0]
    a8 = a8_ref[0]
    bT8 = bT8_ref[...]
    bT = bT_ref[...]

    m = jax.lax.dot_general(
        a8, bT8,
        dimension_numbers=(((1,), (0,)), ((), ())),
        preferred_element_type=jnp.float32,
    )

    a2 = jnp.sum(a * a, axis=1, keepdims=True)
    b2 = jnp.sum(bT * bT, axis=0, keepdims=True)

    t = m + b2
    row_min = jnp.min(t, axis=1, keepdims=True) + a2
    row_min = jnp.maximum(row_min, 0.0)

    u = t + a2

    w = jnp.where(c == 0, 0.5, 1.0)

    @pl.when(jnp.logical_and(c == 0, i == 0))
    def _init_out():
        diff = p3T_ref[...] - bT
        out_ref[0, 0] = jnp.sum(diff * diff) / (_N * 3)

    @pl.when(i == 0)
    def _init_colmin():
        colmin_ref[...] = jnp.full((1, _N), jnp.inf, jnp.float32)

    colmin_ref[...] = jnp.minimum(
        colmin_ref[...], jnp.min(u, axis=0, keepdims=True)
    )

    total = w * jnp.sum(jnp.sqrt(row_min)) / _N
    out_ref[0, 0] = out_ref[0, 0] + total

    @pl.when(i == _NI - 1)
    def _finish_col():
        col_final = jnp.maximum(colmin_ref[...], 0.0)
        col_sum = jnp.sum(jnp.sqrt(col_final)) / _N
        out_ref[0, 0] = out_ref[0, 0] + w * col_sum


def kernel(pc1, pc2):
    a01 = pc1[:2]
    a8 = jnp.zeros((2, _N, 8), jnp.bfloat16)
    a8 = a8.at[:, :, :3].set((-2.0 * a01).astype(jnp.bfloat16))
    bT = pc2.T
    bT8 = jnp.zeros((8, _N), jnp.bfloat16)
    bT8 = bT8.at[:3, :].set(bT.astype(jnp.bfloat16))
    p3T = pc1[3].T

    out = pl.pallas_call(
        _loss_kernel,
        grid=(2, _NI),
        in_specs=[
            pl.BlockSpec((1, _TA, 3), lambda c, i: (c, i, 0)),
            pl.BlockSpec((1, _TA, 8), lambda c, i: (c, i, 0)),
            pl.BlockSpec((8, _N), lambda c, i: (0, 0)),
            pl.BlockSpec((3, _N), lambda c, i: (0, 0)),
            pl.BlockSpec((3, _N), lambda c, i: (0, 0)),
        ],
        out_specs=pl.BlockSpec(memory_space=pltpu.SMEM),
        out_shape=jax.ShapeDtypeStruct((1, 1), jnp.float32),
        scratch_shapes=[pltpu.VMEM((1, _N), jnp.float32)],
        compiler_params=pltpu.CompilerParams(
            dimension_semantics=("arbitrary", "arbitrary"),
        ),
    )(a01, a8, bT8, bT, p3T)
    return out[0, 0]


# speedup vs baseline: 3.3672x; 1.0502x over previous
import jax
import jax.numpy as jnp
from jax.experimental import pallas as pl
from jax.experimental.pallas import tpu as pltpu

_N = 8192
_TA = 1024
_NI = _N // _TA


def _loss_kernel(a_ref, a8_ref, bT8_ref, bT_ref, p3T_ref, out_ref, colmin_ref):
    c = pl.program_id(0)
    i = pl.program_id(1)

    a = a_ref[0]
    a8 = a8_ref[0]
    bT8 = bT8_ref[...]
    bT = bT_ref[...]

    m = jax.lax.dot_general(
        a8, bT8,
        dimension_numbers=(((1,), (0,)), ((), ())),
        preferred_element_type=jnp.float32,
    )

    a2 = jnp.sum(a * a, axis=1, keepdims=True)
    b2 = jnp.sum(bT * bT, axis=0, keepdims=True)

    t = m + b2
    row_min = jnp.min(t, axis=1, keepdims=True) + a2
    row_min = jnp.maximum(row_min, 0.0)

    u = t + a2

    w = jnp.where(c == 0, 0.5, 1.0)

    @pl.when(jnp.logical_and(c == 0, i == 0))
    def _init_out():
        diff = p3T_ref[...] - bT
        out_ref[0, 0] = jnp.sum(diff * diff) / (_N * 3)

    @pl.when(i == 0)
    def _init_colmin():
        colmin_ref[...] = jnp.full((1, _N), jnp.inf, jnp.float32)

    colmin_ref[...] = jnp.minimum(
        colmin_ref[...], jnp.min(u, axis=0, keepdims=True)
    )

    total = w * jnp.sum(jnp.sqrt(row_min)) / _N
    out_ref[0, 0] = out_ref[0, 0] + total

    @pl.when(i == _NI - 1)
    def _finish_col():
        col_final = jnp.maximum(colmin_ref[...], 0.0)
        col_sum = jnp.sum(jnp.sqrt(col_final)) / _N
        out_ref[0, 0] = out_ref[0, 0] + w * col_sum


def kernel(pc1, pc2):
    a01 = pc1[:2]
    a8 = jnp.zeros((2, _N, 8), jnp.bfloat16)
    a8 = a8.at[:, :, :3].set((-2.0 * a01).astype(jnp.bfloat16))
    bT = pc2.T
    bT8 = jnp.zeros((8, _N), jnp.bfloat16)
    bT8 = bT8.at[:3, :].set(bT.astype(jnp.bfloat16))
    p3T = pc1[3].T

    out = pl.pallas_call(
        _loss_kernel,
        grid=(2, _NI),
        in_specs=[
            pl.BlockSpec((1, _TA, 3), lambda c, i: (c, i, 0)),
            pl.BlockSpec((1, _TA, 8), lambda c, i: (c, i, 0)),
            pl.BlockSpec((8, _N), lambda c, i: (0, 0)),
            pl.BlockSpec((3, _N), lambda c, i: (0, 0)),
            pl.BlockSpec((3, _N), lambda c, i: (0, 0)),
        ],
        out_specs=pl.BlockSpec(memory_space=pltpu.SMEM),
        out_shape=jax.ShapeDtypeStruct((1, 1), jnp.float32),
        scratch_shapes=[pltpu.VMEM((1, _N), jnp.float32)],
        compiler_params=pltpu.CompilerParams(
            dimension_semantics=("arbitrary", "arbitrary"),
        ),
    )(a01, a8, bT8, bT, p3T)
    return out[0, 0]
